# baseline (speedup 1.0000x reference)
import jax
import jax.numpy as jnp
from jax import lax
from jax.experimental import pallas as pl
from jax.experimental.pallas import tpu as pltpu
from jax.experimental.pallas import tpu_sc as plsc

N_NODES = 50000
N_EDGES = 1600000
NC = 2
NS = 16
NW = NC * NS
EPW = N_EDGES // NW
GC = 2000
RW = 80
ROWS = N_EDGES // RW
SK = 8
SCH = ROWS // SK
SCPT = (SCH + NW - 1) // NW
NP = 50176
RPT = NP // NS
MB = 2560

import functools


@functools.cache
def _sc_mesh():
    return plsc.VectorSubcoreMesh(core_axis_name="c", subcore_axis_name="s",
                                  num_cores=NC, num_subcores=NS)


def _gather_body(xf_h, src_h, dst_h, dr0_h, dr1_h,
                 xf_v, src_v, dst_v, dr0_v, dr1_v):
    wid = lax.axis_index("s") * NC + lax.axis_index("c")
    pltpu.sync_copy(xf_h, xf_v)
    base = wid * EPW

    def chunk(j, carry):
        off = base + j * GC
        pltpu.sync_copy(src_h.at[pl.ds(off, GC)], src_v)
        pltpu.sync_copy(dst_h.at[pl.ds(off, GC)], dst_v)

        def vec(i, c2):
            s2 = src_v[pl.ds(i * 16, 16)] * 2
            t2 = dst_v[pl.ds(i * 16, 16)] * 2
            dr0_v[pl.ds(i * 16, 16)] = (plsc.load_gather(xf_v, [t2])
                                        - plsc.load_gather(xf_v, [s2]))
            dr1_v[pl.ds(i * 16, 16)] = (plsc.load_gather(xf_v, [t2 + 1])
                                        - plsc.load_gather(xf_v, [s2 + 1]))
            return c2

        lax.fori_loop(0, GC // 16, vec, 0)
        pltpu.sync_copy(dr0_v, dr0_h.at[pl.ds(off, GC)])
        pltpu.sync_copy(dr1_v, dr1_h.at[pl.ds(off, GC)])
        return carry

    lax.fori_loop(0, EPW // GC, chunk, 0)


@functools.cache
def _gather_kernel():
    return pl.kernel(
        _gather_body,
        out_type=[jax.ShapeDtypeStruct((N_EDGES,), jnp.float32),
                  jax.ShapeDtypeStruct((N_EDGES,), jnp.float32)],
        mesh=_sc_mesh(),
        compiler_params=pltpu.CompilerParams(needs_layout_passes=False,
                                             use_tc_tiling_on_sc=False),
        scratch_types=[
            pltpu.VMEM((2 * N_NODES,), jnp.float32),
            pltpu.VMEM((GC,), jnp.int32),
            pltpu.VMEM((GC,), jnp.int32),
            pltpu.VMEM((GC,), jnp.float32),
            pltpu.VMEM((GC,), jnp.float32),
        ],
    )


def _mlp_body(dr0_r, dr1_r, w0c_r, b0c_r, w1t_r, b1c_r, w2t_r, b2c_r, w3r_r,
              b3_r, m0_r, m1_r):
    dr0 = dr0_r[...]
    dr1 = dr1_r[...]
    a = jnp.sqrt(dr0 * dr0 + dr1 * dr1)
    inv = 1.0 / jnp.maximum(a, 1e-12)
    h = jnp.maximum(w0c_r[...] * a + b0c_r[...], 0.0)
    h = jnp.maximum(
        jnp.dot(w1t_r[...], h.astype(jnp.bfloat16),
                preferred_element_type=jnp.float32) + b1c_r[...], 0.0)
    h = jnp.maximum(
        jnp.dot(w2t_r[...], h.astype(jnp.bfloat16),
                preferred_element_type=jnp.float32) + b2c_r[...], 0.0)
    f = (jnp.dot(w3r_r[...], h, preferred_element_type=jnp.float32)
         + b3_r[...])
    m0_r[...] = f * dr0 * inv
    m1_r[...] = f * dr1 * inv


NE = 16000

_mlp = pl.pallas_call(
    _mlp_body,
    grid=(N_EDGES // NE,),
    in_specs=[
        pl.BlockSpec((1, NE), lambda i: (0, i)),
        pl.BlockSpec((1, NE), lambda i: (0, i)),
        pl.BlockSpec((128, 1), lambda i: (0, 0)),
        pl.BlockSpec((128, 1), lambda i: (0, 0)),
        pl.BlockSpec((128, 128), lambda i: (0, 0)),
        pl.BlockSpec((128, 1), lambda i: (0, 0)),
        pl.BlockSpec((128, 128), lambda i: (0, 0)),
        pl.BlockSpec((128, 1), lambda i: (0, 0)),
        pl.BlockSpec((1, 128), lambda i: (0, 0)),
        pl.BlockSpec((1, 1), lambda i: (0, 0)),
    ],
    out_specs=[pl.BlockSpec((1, NE), lambda i: (0, i)),
               pl.BlockSpec((1, NE), lambda i: (0, i))],
    out_shape=[jax.ShapeDtypeStruct((1, N_EDGES), jnp.float32),
               jax.ShapeDtypeStruct((1, N_EDGES), jnp.float32)],
)


def _scatter_body(m0_h, m1_h, dst_h, zeros_h, out_h,
                  dst_v, m0_v, m1_v, srcs_v, obuf_v, acc_sh, sem):
    cid = lax.axis_index("c")
    sid = lax.axis_index("s")
    wid = sid * NC + cid
    pltpu.sync_copy(zeros_h.at[pl.ds(sid * RPT, RPT)], obuf_v)
    pltpu.sync_copy(obuf_v, acc_sh.at[pl.ds(sid * RPT, RPT)])
    for j in range(SK):
        pltpu.sync_copy(zeros_h.at[pl.ds(0, RW)], srcs_v.at[j])
    plsc.subcore_barrier()

    iota16 = lax.iota(jnp.int32, 16)
    zero16 = jnp.zeros((16,), jnp.int32)
    one16 = zero16 + 1

    def chunk(k, carry):
        c = k * NW + wid

        @pl.when(c < SCH)
        def _():
            pltpu.sync_copy(dst_h.at[pl.ds(c * SK, SK)], dst_v)
            pltpu.sync_copy(m0_h.at[pl.ds(c * SK, SK)], m0_v)
            pltpu.sync_copy(m1_h.at[pl.ds(c * SK, SK)], m1_v)
            for j in range(SK):
                for g in range(RW // 16):
                    rows = iota16 + g * 16
                    m0 = m0_v[j, pl.ds(g * 16, 16)]
                    m1 = m1_v[j, pl.ds(g * 16, 16)]
                    plsc.store_scatter(srcs_v.at[j], [rows, zero16], m0)
                    plsc.store_scatter(srcs_v.at[j], [rows, one16], m1)
            descs = []
            for j in range(SK):
                d = pltpu.make_async_copy(srcs_v.at[j],
                                          acc_sh.at[dst_v.at[j]], sem)
                d.start(add=True)
                descs.append(d)
            for d in descs:
                d.wait()

        return carry

    lax.fori_loop(0, SCPT, chunk, 0)
    plsc.subcore_barrier()
    pltpu.sync_copy(acc_sh.at[pl.ds(sid * RPT, RPT)], obuf_v)
    pltpu.sync_copy(obuf_v, out_h.at[cid].at[pl.ds(sid * RPT, RPT)])


@functools.cache
def _scatter_kernel():
    return pl.kernel(
        _scatter_body,
        out_type=jax.ShapeDtypeStruct((NC, NP, 8), jnp.float32),
        mesh=_sc_mesh(),
        compiler_params=pltpu.CompilerParams(needs_layout_passes=False,
                                             use_tc_tiling_on_sc=False),
        scratch_types=[
            pltpu.VMEM((SK, RW), jnp.int32),
            pltpu.VMEM((SK, RW), jnp.float32),
            pltpu.VMEM((SK, RW), jnp.float32),
            pltpu.VMEM((SK, RW, 8), jnp.float32),
            pltpu.VMEM((RPT, 8), jnp.float32),
            pltpu.VMEM_SHARED((NP, 8), jnp.float32),
            pltpu.SemaphoreType.DMA,
        ],
    )


def _combine_body(p0_r, p1_r, gv_r, o_r):
    o_r[...] = (p0_r[...] + p1_r[...]) - gv_r[...]


_combine = pl.pallas_call(
    _combine_body,
    out_shape=jax.ShapeDtypeStruct((800, 125), jnp.float32),
)


def kernel(x, v, edge_index, W0, b0, W1, b1, W2, b2, W3, b3, gamma):
    ei = edge_index.astype(jnp.int32)
    src = ei[0]
    dst = ei[1]
    xf = x.reshape(2 * N_NODES)
    dr0, dr1 = _gather_kernel()(xf, src, dst)
    m0, m1 = _mlp(dr0.reshape(1, N_EDGES), dr1.reshape(1, N_EDGES),
                  W0.reshape(128, 1), b0.reshape(128, 1),
                  W1.T.astype(jnp.bfloat16), b1.reshape(128, 1),
                  W2.T.astype(jnp.bfloat16), b2.reshape(128, 1),
                  W3.reshape(1, 128), b3.reshape(1, 1))
    partial = _scatter_kernel()(m0.reshape(ROWS, RW), m1.reshape(ROWS, RW),
                                dst.reshape(ROWS, RW),
                                jnp.zeros((NP, 8), jnp.float32))
    p0 = partial[0, :N_NODES, :2].reshape(800, 125)
    p1 = partial[1, :N_NODES, :2].reshape(800, 125)
    gv = (gamma * v).reshape(800, 125)
    return _combine(p0, p1, gv).reshape(N_NODES, 2)


# speedup vs baseline: 1.0473x; 1.0473x over previous
import jax
import jax.numpy as jnp
from jax import lax
from jax.experimental import pallas as pl
from jax.experimental.pallas import tpu as pltpu
from jax.experimental.pallas import tpu_sc as plsc

N_NODES = 50000
N_EDGES = 1600000
NC = 2
NS = 16
NW = NC * NS
EPW = N_EDGES // NW
GC = 1000
RW = 80
ROWS = N_EDGES // RW
SK = 8
SCH = ROWS // SK
SCPT = (SCH + NW - 1) // NW
NP = 50176
RPT = NP // NS
MB = 2560

import functools


@functools.cache
def _sc_mesh():
    return plsc.VectorSubcoreMesh(core_axis_name="c", subcore_axis_name="s",
                                  num_cores=NC, num_subcores=NS)


def _gather_body(ne, xf_h, src_h, dst_h, dr0_h, dr1_h,
                 xf_v, src_v, dst_v, dr0_v, dr1_v):
    epw = ne // NW
    wid = lax.axis_index("s") * NC + lax.axis_index("c")
    pltpu.sync_copy(xf_h, xf_v)
    base = wid * epw

    def chunk(j, carry):
        off = base + j * GC
        pltpu.sync_copy(src_h.at[pl.ds(off, GC)], src_v)
        pltpu.sync_copy(dst_h.at[pl.ds(off, GC)], dst_v)

        def vec(i, c2):
            s2 = src_v[pl.ds(i * 16, 16)] * 2
            t2 = dst_v[pl.ds(i * 16, 16)] * 2
            dr0_v[pl.ds(i * 16, 16)] = (plsc.load_gather(xf_v, [t2])
                                        - plsc.load_gather(xf_v, [s2]))
            dr1_v[pl.ds(i * 16, 16)] = (plsc.load_gather(xf_v, [t2 + 1])
                                        - plsc.load_gather(xf_v, [s2 + 1]))
            return c2

        lax.fori_loop(0, GC // 16, vec, 0)
        pltpu.sync_copy(dr0_v, dr0_h.at[pl.ds(off, GC)])
        pltpu.sync_copy(dr1_v, dr1_h.at[pl.ds(off, GC)])
        return carry

    lax.fori_loop(0, epw // GC, chunk, 0)


@functools.cache
def _gather_kernel(ne):
    return pl.kernel(
        functools.partial(_gather_body, ne),
        out_type=[jax.ShapeDtypeStruct((ne,), jnp.float32),
                  jax.ShapeDtypeStruct((ne,), jnp.float32)],
        mesh=_sc_mesh(),
        compiler_params=pltpu.CompilerParams(needs_layout_passes=False,
                                             use_tc_tiling_on_sc=False),
        scratch_types=[
            pltpu.VMEM((2 * N_NODES,), jnp.float32),
            pltpu.VMEM((GC,), jnp.int32),
            pltpu.VMEM((GC,), jnp.int32),
            pltpu.VMEM((GC,), jnp.float32),
            pltpu.VMEM((GC,), jnp.float32),
        ],
    )


def _mlp_body(dr0_r, dr1_r, w0c_r, b0c_r, w1t_r, b1c_r, w2t_r, b2c_r, w3r_r,
              b3_r, m0_r, m1_r):
    dr0 = dr0_r[...]
    dr1 = dr1_r[...]
    a = jnp.sqrt(dr0 * dr0 + dr1 * dr1)
    inv = 1.0 / jnp.maximum(a, 1e-12)
    h = jnp.maximum(w0c_r[...] * a + b0c_r[...], 0.0)
    h = jnp.maximum(
        jnp.dot(w1t_r[...], h.astype(jnp.bfloat16),
                preferred_element_type=jnp.float32) + b1c_r[...], 0.0)
    h = jnp.maximum(
        jnp.dot(w2t_r[...], h.astype(jnp.bfloat16),
                preferred_element_type=jnp.float32) + b2c_r[...], 0.0)
    f = (jnp.dot(w3r_r[...], h, preferred_element_type=jnp.float32)
         + b3_r[...])
    m0_r[...] = f * dr0 * inv
    m1_r[...] = f * dr1 * inv


NE = 16000


@functools.cache
def _mlp_kernel(ne_total):
    return pl.pallas_call(
        _mlp_body,
        grid=(ne_total // NE,),
        in_specs=[
            pl.BlockSpec((1, NE), lambda i: (0, i)),
            pl.BlockSpec((1, NE), lambda i: (0, i)),
            pl.BlockSpec((128, 1), lambda i: (0, 0)),
            pl.BlockSpec((128, 1), lambda i: (0, 0)),
            pl.BlockSpec((128, 128), lambda i: (0, 0)),
            pl.BlockSpec((128, 1), lambda i: (0, 0)),
            pl.BlockSpec((128, 128), lambda i: (0, 0)),
            pl.BlockSpec((128, 1), lambda i: (0, 0)),
            pl.BlockSpec((1, 128), lambda i: (0, 0)),
            pl.BlockSpec((1, 1), lambda i: (0, 0)),
        ],
        out_specs=[pl.BlockSpec((1, NE), lambda i: (0, i)),
                   pl.BlockSpec((1, NE), lambda i: (0, i))],
        out_shape=[jax.ShapeDtypeStruct((1, ne_total), jnp.float32),
                   jax.ShapeDtypeStruct((1, ne_total), jnp.float32)],
    )


def _scatter_body(rows, m0_h, m1_h, dst_h, zeros_h, out_h,
                  dst_v, m0_v, m1_v, srcs_v, obuf_v, acc_sh, sem):
    sch = rows // SK
    scpt = (sch + NW - 1) // NW
    cid = lax.axis_index("c")
    sid = lax.axis_index("s")
    wid = sid * NC + cid
    pltpu.sync_copy(zeros_h.at[pl.ds(sid * RPT, RPT)], obuf_v)
    pltpu.sync_copy(obuf_v, acc_sh.at[pl.ds(sid * RPT, RPT)])
    for j in range(SK):
        pltpu.sync_copy(zeros_h.at[pl.ds(0, RW)], srcs_v.at[j])
    plsc.subcore_barrier()

    iota16 = lax.iota(jnp.int32, 16)
    zero16 = jnp.zeros((16,), jnp.int32)
    one16 = zero16 + 1

    def chunk(k, carry):
        c = k * NW + wid

        @pl.when(c < sch)
        def _():
            pltpu.sync_copy(dst_h.at[pl.ds(c * SK, SK)], dst_v)
            pltpu.sync_copy(m0_h.at[pl.ds(c * SK, SK)], m0_v)
            pltpu.sync_copy(m1_h.at[pl.ds(c * SK, SK)], m1_v)
            for j in range(SK):
                for g in range(RW // 16):
                    rows = iota16 + g * 16
                    m0 = m0_v[j, pl.ds(g * 16, 16)]
                    m1 = m1_v[j, pl.ds(g * 16, 16)]
                    plsc.store_scatter(srcs_v.at[j], [rows, zero16], m0)
                    plsc.store_scatter(srcs_v.at[j], [rows, one16], m1)
            descs = []
            for j in range(SK):
                d = pltpu.make_async_copy(srcs_v.at[j],
                                          acc_sh.at[dst_v.at[j]], sem)
                d.start(add=True)
                descs.append(d)
            for d in descs:
                d.wait()

        return carry

    lax.fori_loop(0, scpt, chunk, 0)
    plsc.subcore_barrier()
    pltpu.sync_copy(acc_sh.at[pl.ds(sid * RPT, RPT)], obuf_v)
    pltpu.sync_copy(obuf_v, out_h.at[cid].at[pl.ds(sid * RPT, RPT)])


@functools.cache
def _scatter_kernel(rows):
    return pl.kernel(
        functools.partial(_scatter_body, rows),
        out_type=jax.ShapeDtypeStruct((NC, NP, 8), jnp.float32),
        mesh=_sc_mesh(),
        compiler_params=pltpu.CompilerParams(needs_layout_passes=False,
                                             use_tc_tiling_on_sc=False),
        scratch_types=[
            pltpu.VMEM((SK, RW), jnp.int32),
            pltpu.VMEM((SK, RW), jnp.float32),
            pltpu.VMEM((SK, RW), jnp.float32),
            pltpu.VMEM((SK, RW, 8), jnp.float32),
            pltpu.VMEM((RPT, 8), jnp.float32),
            pltpu.VMEM_SHARED((NP, 8), jnp.float32),
            pltpu.SemaphoreType.DMA,
        ],
    )


def _combine_body(p0_r, p1_r, p2_r, p3_r, gv_r, o_r):
    o_r[...] = (p0_r[...] + p1_r[...] + p2_r[...] + p3_r[...]) - gv_r[...]


_combine = pl.pallas_call(
    _combine_body,
    out_shape=jax.ShapeDtypeStruct((800, 125), jnp.float32),
)


def kernel(x, v, edge_index, W0, b0, W1, b1, W2, b2, W3, b3, gamma):
    ei = edge_index.astype(jnp.int32)
    src = ei[0]
    dst = ei[1]
    xf = x.reshape(2 * N_NODES)
    EH = N_EDGES // 2
    RH = EH // RW
    zeros = jnp.zeros((NP, 8), jnp.float32)
    w_args = (W0.reshape(128, 1), b0.reshape(128, 1),
              W1.T.astype(jnp.bfloat16), b1.reshape(128, 1),
              W2.T.astype(jnp.bfloat16), b2.reshape(128, 1),
              W3.reshape(1, 128), b3.reshape(1, 1))
    parts = []
    drs = []
    for h in range(2):
        s_h = src[h * EH:(h + 1) * EH]
        d_h = dst[h * EH:(h + 1) * EH]
        dr0, dr1 = _gather_kernel(EH)(xf, s_h, d_h)
        drs.append((dr0, dr1, d_h))
    for h in range(2):
        dr0, dr1, d_h = drs[h]
        m0, m1 = _mlp_kernel(EH)(dr0.reshape(1, EH), dr1.reshape(1, EH),
                                 *w_args)
        partial = _scatter_kernel(RH)(m0.reshape(RH, RW), m1.reshape(RH, RW),
                                      d_h.reshape(RH, RW), zeros)
        parts.append(partial)
    ps = [parts[h][c, :N_NODES, :2].reshape(800, 125)
          for h in range(2) for c in range(NC)]
    gv = (gamma * v).reshape(800, 125)
    return _combine(*ps, gv).reshape(N_NODES, 2)


# speedup vs baseline: 1.1623x; 1.1098x over previous
import jax
import jax.numpy as jnp
from jax import lax
from jax.experimental import pallas as pl
from jax.experimental.pallas import tpu as pltpu
from jax.experimental.pallas import tpu_sc as plsc

N_NODES = 50000
N_EDGES = 1600000
NC = 2
NS = 16
NW = NC * NS
EPW = N_EDGES // NW
GC = 2000
RW = 80
ROWS = N_EDGES // RW
SK = 8
SCH = ROWS // SK
SCPT = (SCH + NW - 1) // NW
NP = 50176
RPT = NP // NS
MB = 2560

import functools


@functools.cache
def _sc_mesh():
    return plsc.VectorSubcoreMesh(core_axis_name="c", subcore_axis_name="s",
                                  num_cores=NC, num_subcores=NS)


def _gather_body(ne, xf_h, src_h, dst_h, dr0_h, dr1_h,
                 xf_v, src_v, dst_v, dr0_v, dr1_v):
    epw = ne // NW
    wid = lax.axis_index("s") * NC + lax.axis_index("c")
    pltpu.sync_copy(xf_h, xf_v)
    base = wid * epw

    def chunk(j, carry):
        off = base + j * GC
        pltpu.sync_copy(src_h.at[pl.ds(off, GC)], src_v)
        pltpu.sync_copy(dst_h.at[pl.ds(off, GC)], dst_v)

        def vec(i, c2):
            s2 = src_v[pl.ds(i * 16, 16)] * 2
            t2 = dst_v[pl.ds(i * 16, 16)] * 2
            dr0_v[pl.ds(i * 16, 16)] = (plsc.load_gather(xf_v, [t2])
                                        - plsc.load_gather(xf_v, [s2]))
            dr1_v[pl.ds(i * 16, 16)] = (plsc.load_gather(xf_v, [t2 + 1])
                                        - plsc.load_gather(xf_v, [s2 + 1]))
            return c2

        lax.fori_loop(0, GC // 16, vec, 0)
        pltpu.sync_copy(dr0_v, dr0_h.at[pl.ds(off, GC)])
        pltpu.sync_copy(dr1_v, dr1_h.at[pl.ds(off, GC)])
        return carry

    lax.fori_loop(0, epw // GC, chunk, 0)


@functools.cache
def _gather_kernel(ne):
    return pl.kernel(
        functools.partial(_gather_body, ne),
        out_type=[jax.ShapeDtypeStruct((ne,), jnp.float32),
                  jax.ShapeDtypeStruct((ne,), jnp.float32)],
        mesh=_sc_mesh(),
        compiler_params=pltpu.CompilerParams(needs_layout_passes=False,
                                             use_tc_tiling_on_sc=False),
        scratch_types=[
            pltpu.VMEM((2 * N_NODES,), jnp.float32),
            pltpu.VMEM((GC,), jnp.int32),
            pltpu.VMEM((GC,), jnp.int32),
            pltpu.VMEM((GC,), jnp.float32),
            pltpu.VMEM((GC,), jnp.float32),
        ],
    )


def _mlp_body(dr0_r, dr1_r, w0c_r, b0c_r, w1t_r, b1c_r, w2t_r, b2c_r, w3r_r,
              b3_r, m0_r, m1_r):
    dr0 = dr0_r[...]
    dr1 = dr1_r[...]
    a = jnp.sqrt(dr0 * dr0 + dr1 * dr1)
    inv = 1.0 / jnp.maximum(a, 1e-12)
    h = jnp.maximum(w0c_r[...] * a + b0c_r[...], 0.0)
    h = jnp.maximum(
        jnp.dot(w1t_r[...], h.astype(jnp.bfloat16),
                preferred_element_type=jnp.float32) + b1c_r[...], 0.0)
    h = jnp.maximum(
        jnp.dot(w2t_r[...], h.astype(jnp.bfloat16),
                preferred_element_type=jnp.float32) + b2c_r[...], 0.0)
    f = (jnp.dot(w3r_r[...], h, preferred_element_type=jnp.float32)
         + b3_r[...])
    m0_r[...] = f * dr0 * inv
    m1_r[...] = f * dr1 * inv


NE = 16000


@functools.cache
def _mlp_kernel(ne_total):
    return pl.pallas_call(
        _mlp_body,
        grid=(ne_total // NE,),
        in_specs=[
            pl.BlockSpec((1, NE), lambda i: (0, i)),
            pl.BlockSpec((1, NE), lambda i: (0, i)),
            pl.BlockSpec((128, 1), lambda i: (0, 0)),
            pl.BlockSpec((128, 1), lambda i: (0, 0)),
            pl.BlockSpec((128, 128), lambda i: (0, 0)),
            pl.BlockSpec((128, 1), lambda i: (0, 0)),
            pl.BlockSpec((128, 128), lambda i: (0, 0)),
            pl.BlockSpec((128, 1), lambda i: (0, 0)),
            pl.BlockSpec((1, 128), lambda i: (0, 0)),
            pl.BlockSpec((1, 1), lambda i: (0, 0)),
        ],
        out_specs=[pl.BlockSpec((1, NE), lambda i: (0, i)),
                   pl.BlockSpec((1, NE), lambda i: (0, i))],
        out_shape=[jax.ShapeDtypeStruct((1, ne_total), jnp.float32),
                   jax.ShapeDtypeStruct((1, ne_total), jnp.float32)],
    )


def _scatter_body(rows, m0_h, m1_h, dst_h, zeros_h, out_h,
                  dst_v, m0_v, m1_v, srcs_v, obuf_v, acc_sh, sem):
    sch = rows // SK
    scpt = (sch + NW - 1) // NW
    cid = lax.axis_index("c")
    sid = lax.axis_index("s")
    wid = sid * NC + cid
    pltpu.sync_copy(zeros_h.at[pl.ds(sid * RPT, RPT)], obuf_v)
    pltpu.sync_copy(obuf_v, acc_sh.at[pl.ds(sid * RPT, RPT)])
    for j in range(SK):
        pltpu.sync_copy(zeros_h.at[pl.ds(0, RW)], srcs_v.at[j])
    plsc.subcore_barrier()

    iota16 = lax.iota(jnp.int32, 16)
    zero16 = jnp.zeros((16,), jnp.int32)
    one16 = zero16 + 1

    def chunk(k, carry):
        c = k * NW + wid

        @pl.when(c < sch)
        def _():
            pltpu.sync_copy(dst_h.at[pl.ds(c * SK, SK)], dst_v)
            pltpu.sync_copy(m0_h.at[pl.ds(c * SK, SK)], m0_v)
            pltpu.sync_copy(m1_h.at[pl.ds(c * SK, SK)], m1_v)
            for j in range(SK):
                for g in range(RW // 16):
                    rows = iota16 + g * 16
                    m0 = m0_v[j, pl.ds(g * 16, 16)]
                    m1 = m1_v[j, pl.ds(g * 16, 16)]
                    plsc.store_scatter(srcs_v.at[j], [rows, zero16], m0)
                    plsc.store_scatter(srcs_v.at[j], [rows, one16], m1)
            descs = []
            for j in range(SK):
                d = pltpu.make_async_copy(srcs_v.at[j],
                                          acc_sh.at[dst_v.at[j]], sem)
                d.start(add=True)
                descs.append(d)
            for d in descs:
                d.wait()

        return carry

    lax.fori_loop(0, scpt, chunk, 0)
    plsc.subcore_barrier()
    pltpu.sync_copy(acc_sh.at[pl.ds(sid * RPT, RPT)], obuf_v)
    pltpu.sync_copy(obuf_v, out_h.at[cid].at[pl.ds(sid * RPT, RPT)])


@functools.cache
def _scatter_kernel(rows):
    return pl.kernel(
        functools.partial(_scatter_body, rows),
        out_type=jax.ShapeDtypeStruct((NC, NP, 8), jnp.float32),
        mesh=_sc_mesh(),
        compiler_params=pltpu.CompilerParams(needs_layout_passes=False,
                                             use_tc_tiling_on_sc=False),
        scratch_types=[
            pltpu.VMEM((SK, RW), jnp.int32),
            pltpu.VMEM((SK, RW), jnp.float32),
            pltpu.VMEM((SK, RW), jnp.float32),
            pltpu.VMEM((SK, RW, 8), jnp.float32),
            pltpu.VMEM((RPT, 8), jnp.float32),
            pltpu.VMEM_SHARED((NP, 8), jnp.float32),
            pltpu.SemaphoreType.DMA,
        ],
    )


def _combine_body(p0_r, p1_r, p2_r, p3_r, gv_r, o_r):
    o_r[...] = (p0_r[...] + p1_r[...] + p2_r[...] + p3_r[...]) - gv_r[...]


_combine = pl.pallas_call(
    _combine_body,
    out_shape=jax.ShapeDtypeStruct((800, 125), jnp.float32),
)


def kernel(x, v, edge_index, W0, b0, W1, b1, W2, b2, W3, b3, gamma):
    ei = edge_index.astype(jnp.int32)
    src = ei[0]
    dst = ei[1]
    xf = x.reshape(2 * N_NODES)
    EHS = (768000, 832000)
    zeros = jnp.zeros((NP, 8), jnp.float32)
    w_args = (W0.reshape(128, 1), b0.reshape(128, 1),
              W1.T.astype(jnp.bfloat16), b1.reshape(128, 1),
              W2.T.astype(jnp.bfloat16), b2.reshape(128, 1),
              W3.reshape(1, 128), b3.reshape(1, 1))
    parts = []
    drs = []
    off = 0
    for h in range(2):
        EH = EHS[h]
        s_h = src[off:off + EH]
        d_h = dst[off:off + EH]
        off += EH
        dr0, dr1 = _gather_kernel(EH)(xf, s_h, d_h)
        drs.append((dr0, dr1, d_h))
    for h in range(2):
        EH = EHS[h]
        RH = EH // RW
        dr0, dr1, d_h = drs[h]
        m0, m1 = _mlp_kernel(EH)(dr0.reshape(1, EH), dr1.reshape(1, EH),
                                 *w_args)
        partial = _scatter_kernel(RH)(m0.reshape(RH, RW), m1.reshape(RH, RW),
                                      d_h.reshape(RH, RW), zeros)
        parts.append(partial)
    ps = [parts[h][c, :N_NODES, :2].reshape(800, 125)
          for h in range(2) for c in range(NC)]
    gv = (gamma * v).reshape(800, 125)
    return _combine(*ps, gv).reshape(N_NODES, 2)
